# baseline (device time: 2948040 ns/iter reference)
import jax
import jax.numpy as jnp
import numpy as np
from jax import lax
from jax.experimental import pallas as pl
from jax.experimental.pallas import tpu as pltpu

N_DEV = 16


def _ring_order():
    try:
        import distributed_mesh_v7x as dm

        mesh = dm.get_mesh("i", world_size=N_DEV)
        devs = list(np.asarray(mesh.devices).ravel())
        coords = [tuple(d.coords) for d in devs]
        if len(coords) != N_DEV or len(set(coords)) != N_DEV:
            raise ValueError("bad coords")

        def adjacent(a, b):
            return sum(abs(u - v) for u, v in zip(a, b)) == 1

        adj = [
            [adjacent(coords[i], coords[j]) for j in range(N_DEV)]
            for i in range(N_DEV)
        ]
        path = [0]
        used = [False] * N_DEV
        used[0] = True

        def bt():
            if len(path) == N_DEV:
                return adj[path[-1]][0]
            last = path[-1]
            for v in range(N_DEV):
                if not used[v] and adj[last][v]:
                    used[v] = True
                    path.append(v)
                    if bt():
                        return True
                    path.pop()
                    used[v] = False
            return False

        if not bt():
            raise ValueError("no hamiltonian cycle")
        order = np.array(path, dtype=np.int32)
    except Exception:
        order = np.arange(N_DEV, dtype=np.int32)
    pos = np.empty(N_DEV, dtype=np.int32)
    pos[order] = np.arange(N_DEV, dtype=np.int32)
    return order, pos


def kernel(x, w_mat):
    m, k_per = x.shape
    _, n = w_mat.shape
    m_ch = m // N_DEV
    n_hops = 2 * (N_DEV - 1)

    order_np, pos_np = _ring_order()
    my_l = lax.axis_index("i")
    my_r = jnp.asarray(pos_np)[my_l]
    right_l = jnp.asarray(order_np)[(my_r + 1) % N_DEV]
    left_l = jnp.asarray(order_np)[(my_r - 1) % N_DEV]
    meta = jnp.stack([my_r, left_l, right_l]).astype(jnp.int32)

    def body(meta_ref, x_ref, w_ref, out_ref, comm_ref, send_sems, recv_sems,
             credit_sem, copy_sem):
        my_r = meta_ref[0]
        left = meta_ref[1]
        right = meta_ref[2]

        def partial(c):
            return jnp.dot(
                x_ref[pl.ds(c * m_ch, m_ch), :],
                w_ref[:, :],
                preferred_element_type=jnp.float32,
            )

        barrier_sem = pltpu.get_barrier_semaphore()
        for nbr in (left, right):
            pl.semaphore_signal(
                barrier_sem, inc=1, device_id=(nbr,),
                device_id_type=pl.DeviceIdType.MESH,
            )
        pl.semaphore_wait(barrier_sem, 2)

        comm_ref[0, :, :] = partial((my_r - 1) % N_DEV)

        for h in range(n_hops):
            s_slot = h % 2
            r_slot = (h + 1) % 2
            if h >= 1:
                pl.semaphore_wait(credit_sem, 1)
            rdma = pltpu.make_async_remote_copy(
                src_ref=comm_ref.at[s_slot],
                dst_ref=comm_ref.at[r_slot],
                send_sem=send_sems.at[s_slot],
                recv_sem=recv_sems.at[r_slot],
                device_id=(right,),
                device_id_type=pl.DeviceIdType.MESH,
            )
            rdma.start()
            rdma.wait()

            if h < N_DEV - 1:
                c = (my_r - 2 - h) % N_DEV
                comm_ref[r_slot, :, :] = comm_ref[r_slot, :, :] + partial(c)
                if h == N_DEV - 2:
                    cp = pltpu.make_async_copy(
                        comm_ref.at[r_slot],
                        out_ref.at[pl.ds(my_r * m_ch, m_ch), :],
                        copy_sem,
                    )
                    cp.start()
                    cp.wait()
            else:
                t = h - (N_DEV - 1)
                c = (my_r - 1 - t) % N_DEV
                cp = pltpu.make_async_copy(
                    comm_ref.at[r_slot],
                    out_ref.at[pl.ds(c * m_ch, m_ch), :],
                    copy_sem,
                )
                cp.start()
                cp.wait()

            if h < n_hops - 1:
                pl.semaphore_signal(
                    credit_sem, inc=1, device_id=(left,),
                    device_id_type=pl.DeviceIdType.MESH,
                )

    return pl.pallas_call(
        body,
        out_shape=jax.ShapeDtypeStruct((m, n), jnp.float32),
        in_specs=[
            pl.BlockSpec(memory_space=pltpu.SMEM),
            pl.BlockSpec(memory_space=pltpu.VMEM),
            pl.BlockSpec(memory_space=pltpu.VMEM),
        ],
        out_specs=pl.BlockSpec(memory_space=pl.ANY),
        scratch_shapes=[
            pltpu.VMEM((2, m_ch, n), jnp.float32),
            pltpu.SemaphoreType.DMA((2,)),
            pltpu.SemaphoreType.DMA((2,)),
            pltpu.SemaphoreType.REGULAR,
            pltpu.SemaphoreType.DMA,
        ],
        compiler_params=pltpu.CompilerParams(collective_id=0),
    )(meta, x, w_mat)
